# Initial kernel scaffold; baseline (speedup 1.0000x reference)
import functools

import jax
import jax.numpy as jnp
import numpy as np
from jax.experimental import pallas as pl
from jax.experimental.pallas import tpu as pltpu

_DOWN = [0, 1, 2, 4, 5, 7, 8, 10, 11]
_LH = list(range(25, 40))
_RH = list(range(40, 55))
_UP = [3, 6, 9, 12, 13, 14, 15, 16, 17, 18, 19, 20, 21, 22, 23, 24]

_JC = 3
_RC = 6
_K = 512
_D = 64

_HIGH = jax.lax.Precision.HIGHEST


def _scatter_cols(w, joints, width_per_joint, out_cols):
    out = jnp.zeros(w.shape[:-1] + (out_cols,), dtype=w.dtype)
    for k, j in enumerate(joints):
        out = out.at[..., j * width_per_joint:(j + 1) * width_per_joint].set(
            w[..., k * width_per_joint:(k + 1) * width_per_joint])
    return out


def _prep_body(up_cb, dn_cb, lh_cb, rh_cb,
               up_wp, up_wr, dn_wp, dn_wr, dn_ws, lh_wr, rh_wr,
               up_bp, up_br, dn_bp, dn_br, dn_bs, lh_br, rh_br,
               p_up, r_up, p_dn, r_dn, s_dn, r_lh, r_rh):
    p_up[...] = jax.lax.dot(up_cb[...], up_wp[...], precision=_HIGH) + up_bp[...]
    r_up[...] = jax.lax.dot(up_cb[...], up_wr[...], precision=_HIGH) + up_br[...]
    p_dn[...] = jax.lax.dot(dn_cb[...], dn_wp[...], precision=_HIGH) + dn_bp[...]
    r_dn[...] = jax.lax.dot(dn_cb[...], dn_wr[...], precision=_HIGH) + dn_br[...]
    s_dn[...] = jax.lax.dot(dn_cb[...], dn_ws[...], precision=_HIGH) + dn_bs[...]
    r_lh[...] = jax.lax.dot(lh_cb[...], lh_wr[...], precision=_HIGH) + lh_br[...]
    r_rh[...] = jax.lax.dot(rh_cb[...], rh_wr[...], precision=_HIGH) + rh_br[...]


def _main_body(x, xr, xs, wx, wr, ws, eb,
               up_cb, dn_cb, lh_cb, rh_cb,
               p_up, r_up, p_dn, r_dn, s_dn, r_lh, r_rh,
               xout, xrout, xshift, loss, *, tile, n_rows):
    xt = x[...]
    xrt = xr[...]
    xst = xs[...]
    z_all = (jax.lax.dot(xt, wx[...], precision=_HIGH)
             + jax.lax.dot(xrt, wr[...], precision=_HIGH)
             + jax.lax.dot(xst, ws[...], precision=_HIGH)
             + eb[...])

    iota = jax.lax.broadcasted_iota(jnp.int32, (tile, _K), 1)
    dmin_total = jnp.zeros((), jnp.float32)
    oh = []
    for p, cb_ref in enumerate((up_cb, dn_cb, lh_cb, rh_cb)):
        cb = cb_ref[...]
        z = z_all[:, p * _D:(p + 1) * _D]
        zn = jnp.sum(z * z, axis=1, keepdims=True)
        cn = jnp.sum(cb * cb, axis=1)
        scores = jax.lax.dot_general(z, cb, (((1,), (1,)), ((), ())),
                                     precision=_HIGH)
        d = zn - 2.0 * scores + cn[None, :]
        idx = jnp.argmin(d, axis=1).astype(jnp.int32)
        dmin_total += jnp.sum(jnp.min(d, axis=1))
        oh.append((iota == idx[:, None]).astype(jnp.float32))
    oh_up, oh_dn, oh_lh, oh_rh = oh

    pos_up = jax.lax.dot(oh_up, p_up[...], precision=_HIGH)
    pos_dn = jax.lax.dot(oh_dn, p_dn[...], precision=_HIGH)
    xout[...] = jnp.concatenate(
        [pos_up[:, :36] + pos_dn, pos_up[:, 36:75], xt[:, 75:165]], axis=1)

    rot_up = jax.lax.dot(oh_up, r_up[...], precision=_HIGH)
    rot_dn = jax.lax.dot(oh_dn, r_dn[...], precision=_HIGH)
    rot_lh = jax.lax.dot(oh_lh, r_lh[...], precision=_HIGH)
    rot_rh = jax.lax.dot(oh_rh, r_rh[...], precision=_HIGH)
    xrout[...] = jnp.concatenate(
        [rot_up[:, :72] + rot_dn, rot_up[:, 72:150], rot_lh, rot_rh], axis=1)

    xshift[...] = jax.lax.dot(oh_dn, s_dn[...], precision=_HIGH)

    @pl.when(pl.program_id(0) == 0)
    def _():
        loss[...] = jnp.zeros_like(loss)
    loss[0, 0] += (1.25 / (n_rows * _D)) * dmin_total


def kernel(x, xrot, xshift, up_encW, up_encb, up_codebook, up_decW, up_decb,
           down_encW, down_encb, down_codebook, down_decW, down_decb,
           lhand_encW, lhand_encb, lhand_codebook, lhand_decW, lhand_decb,
           rhand_encW, rhand_encb, rhand_codebook, rhand_decW, rhand_decb):
    b, t, c = x.shape
    crot = xrot.shape[-1]
    n = b * t
    x2 = x.reshape(n, c)
    xr2 = xrot.reshape(n, crot)
    xs2 = xshift.reshape(n, 3)

    parts = {
        'up': (_UP, up_encW, up_encb),
        'down': (_DOWN, down_encW, down_encb),
        'lhand': (_LH, lhand_encW, lhand_encb),
        'rhand': (_RH, rhand_encW, rhand_encb),
    }

    wx = jnp.zeros((c, 4 * _D), jnp.float32)
    wr = jnp.zeros((crot, 4 * _D), jnp.float32)
    ws = jnp.zeros((3, 4 * _D), jnp.float32)
    eb = jnp.zeros((1, 4 * _D), jnp.float32)
    for p, (name, (joints, encW, encb)) in enumerate(parts.items()):
        nj = len(joints)
        cp = nj * _JC
        cr = nj * _RC
        for k, j in enumerate(joints):
            wx = wx.at[j * _JC:(j + 1) * _JC, p * _D:(p + 1) * _D].set(
                encW[k * _JC:(k + 1) * _JC])
            wr = wr.at[j * _RC:(j + 1) * _RC, p * _D:(p + 1) * _D].set(
                encW[cp + k * _RC:cp + (k + 1) * _RC])
        if name == 'down':
            ws = ws.at[:, p * _D:(p + 1) * _D].set(encW[cp + cr:cp + cr + 3])
        eb = eb.at[0, p * _D:(p + 1) * _D].set(encb)

    up_cp = len(_UP) * _JC
    dn_cp = len(_DOWN) * _JC
    up_wp = _scatter_cols(up_decW[:, :up_cp], _UP, _JC, 75)
    up_bp = _scatter_cols(up_decb[None, :up_cp], _UP, _JC, 75)
    up_wr = _scatter_cols(up_decW[:, up_cp:up_cp + len(_UP) * _RC], _UP, _RC,
                          150)
    up_br = _scatter_cols(up_decb[None, up_cp:up_cp + len(_UP) * _RC], _UP,
                          _RC, 150)
    dn_wp = _scatter_cols(down_decW[:, :dn_cp], _DOWN, _JC, 36)
    dn_bp = _scatter_cols(down_decb[None, :dn_cp], _DOWN, _JC, 36)
    dn_wr = _scatter_cols(down_decW[:, dn_cp:dn_cp + len(_DOWN) * _RC],
                          _DOWN, _RC, 72)
    dn_br = _scatter_cols(down_decb[None, dn_cp:dn_cp + len(_DOWN) * _RC],
                          _DOWN, _RC, 72)
    dn_ws_w = down_decW[:, dn_cp + len(_DOWN) * _RC:]
    dn_bs = down_decb[None, dn_cp + len(_DOWN) * _RC:]
    lh_cp = len(_LH) * _JC
    lh_wr = lhand_decW[:, lh_cp:lh_cp + len(_LH) * _RC]
    lh_br = lhand_decb[None, lh_cp:lh_cp + len(_LH) * _RC]
    rh_cp = len(_RH) * _JC
    rh_wr = rhand_decW[:, rh_cp:rh_cp + len(_RH) * _RC]
    rh_br = rhand_decb[None, rh_cp:rh_cp + len(_RH) * _RC]

    tbl_shapes = [
        jax.ShapeDtypeStruct((_K, 75), jnp.float32),
        jax.ShapeDtypeStruct((_K, 150), jnp.float32),
        jax.ShapeDtypeStruct((_K, 36), jnp.float32),
        jax.ShapeDtypeStruct((_K, 72), jnp.float32),
        jax.ShapeDtypeStruct((_K, 3), jnp.float32),
        jax.ShapeDtypeStruct((_K, 90), jnp.float32),
        jax.ShapeDtypeStruct((_K, 90), jnp.float32),
    ]
    tables = pl.pallas_call(
        _prep_body,
        out_shape=tbl_shapes,
    )(up_codebook, down_codebook, lhand_codebook, rhand_codebook,
      up_wp, up_wr, dn_wp, dn_wr, dn_ws_w, lh_wr, rh_wr,
      up_bp, up_br, dn_bp, dn_br, dn_bs, lh_br, rh_br)
    p_up, r_up, p_dn, r_dn, s_dn, r_lh, r_rh = tables

    tile = 256
    grid = n // tile
    row_spec = lambda w: pl.BlockSpec((tile, w), lambda i: (i, 0))
    full = lambda a: pl.BlockSpec(a.shape, lambda i: (0,) * a.ndim)

    out_shapes = [
        jax.ShapeDtypeStruct((n, c), jnp.float32),
        jax.ShapeDtypeStruct((n, crot), jnp.float32),
        jax.ShapeDtypeStruct((n, 3), jnp.float32),
        jax.ShapeDtypeStruct((1, 1), jnp.float32),
    ]
    outs = pl.pallas_call(
        functools.partial(_main_body, tile=tile, n_rows=n),
        grid=(grid,),
        in_specs=[row_spec(c), row_spec(crot), row_spec(3),
                  full(wx), full(wr), full(ws), full(eb),
                  full(up_codebook), full(down_codebook),
                  full(lhand_codebook), full(rhand_codebook),
                  full(p_up), full(r_up), full(p_dn), full(r_dn),
                  full(s_dn), full(r_lh), full(r_rh)],
        out_specs=[row_spec(c), row_spec(crot), row_spec(3),
                   pl.BlockSpec((1, 1), lambda i: (0, 0))],
        out_shape=out_shapes,
    )(x2, xr2, xs2, wx, wr, ws, eb,
      up_codebook, down_codebook, lhand_codebook, rhand_codebook,
      p_up, r_up, p_dn, r_dn, s_dn, r_lh, r_rh)
    xout2, xrout2, xshift2, loss = outs

    return (xout2.reshape(b, t, c), xrout2.reshape(b, t, crot),
            xshift2.reshape(b, t, 3), loss[0, 0])


# speedup vs baseline: 2.4878x; 2.4878x over previous
import functools

import jax
import jax.numpy as jnp
import numpy as np
from jax.experimental import pallas as pl
from jax.experimental.pallas import tpu as pltpu

_DOWN = [0, 1, 2, 4, 5, 7, 8, 10, 11]
_LH = list(range(25, 40))
_RH = list(range(40, 55))
_UP = [3, 6, 9, 12, 13, 14, 15, 16, 17, 18, 19, 20, 21, 22, 23, 24]

_JC = 3
_RC = 6
_K = 512
_D = 64

_PREC = jax.lax.Precision.DEFAULT

_PARTS = ('up', 'down', 'lhand', 'rhand')
_JOINTS = {'up': _UP, 'down': _DOWN, 'lhand': _LH, 'rhand': _RH}


def _part_layout():
    offs, cps, off = {}, {}, 0
    for name in _PARTS:
        nj = len(_JOINTS[name])
        cin = nj * (_JC + _RC) + (3 if name == 'down' else 0)
        offs[name] = off
        cps[name] = nj * _JC
        off += cin
    return offs, cps, off


def _enc_row_maps():
    offs, cps, total = _part_layout()
    j2p = {}
    for name in _PARTS:
        for k, j in enumerate(_JOINTS[name]):
            j2p[j] = (name, k)
    xrows = np.zeros(55 * _JC, np.int32)
    rrows = np.zeros(55 * _RC, np.int32)
    for j in range(55):
        name, k = j2p[j]
        for cc in range(_JC):
            xrows[j * _JC + cc] = offs[name] + k * _JC + cc
        for cc in range(_RC):
            rrows[j * _RC + cc] = offs[name] + cps[name] + k * _RC + cc
    srows = np.array([offs['down'] + cps['down'] + len(_DOWN) * _RC + cc
                      for cc in range(3)], np.int32)
    return xrows, rrows, srows


def _dec_col_map(name, widths):
    joints = _JOINTS[name]
    nj = len(joints)
    cp = nj * _JC
    pos_w, rot_w = widths
    src = np.zeros(pos_w + rot_w + (3 if name == 'down' else 0), np.int32)
    msk = np.zeros_like(src, np.float32)
    for k, j in enumerate(joints):
        for cc in range(_JC):
            src[j * _JC + cc] = k * _JC + cc
            msk[j * _JC + cc] = 1.0
        for cc in range(_RC):
            src[pos_w + j * _RC + cc] = cp + k * _RC + cc
            msk[pos_w + j * _RC + cc] = 1.0
    if name == 'down':
        for cc in range(3):
            src[pos_w + rot_w + cc] = cp + nj * _RC + cc
            msk[pos_w + rot_w + cc] = 1.0
    return src, msk


def _prep_body(up_cb, dn_cb, lh_cb, rh_cb,
               up_w, dn_w, lh_w, rh_w, up_b, dn_b, lh_b, rh_b,
               t_up, t_dn, t_lh, t_rh,
               m2_up, m2_dn, m2_lh, m2_rh, cn):
    cbs = (up_cb[...], dn_cb[...], lh_cb[...], rh_cb[...])
    for cb, w, bias, tbl in zip(
            cbs, (up_w, dn_w, lh_w, rh_w), (up_b, dn_b, lh_b, rh_b),
            (t_up, t_dn, t_lh, t_rh)):
        tbl[...] = jax.lax.dot(cb, w[...], precision=_PREC) + bias[...]
    for p, (cb, m2) in enumerate(zip(cbs, (m2_up, m2_dn, m2_lh, m2_rh))):
        m2[...] = -2.0 * cb
        cn[p, :] = jnp.sum(cb * cb, axis=1)


def _main_body(x, xr, xs, wx, wr, ws, eb,
               m2_up, m2_dn, m2_lh, m2_rh, cn,
               t_up, t_dn, t_lh, t_rh,
               xout, xrout, xshift, loss, *, tile, n_rows):
    xt = x[0]
    xrt = xr[0]
    xst = xs[0]
    z_all = (jax.lax.dot(xt, wx[...], precision=_PREC)
             + jax.lax.dot(xrt, wr[...], precision=_PREC)
             + jax.lax.dot(xst, ws[...], precision=_PREC)
             + eb[...])

    cn_all = cn[...]
    dmin_total = jnp.sum(z_all * z_all)
    iota = jax.lax.broadcasted_iota(jnp.int32, (tile, _K), 1).astype(
        jnp.float32)
    dec = []
    for p, (m2cb, tbl) in enumerate(
            ((m2_up, t_up), (m2_dn, t_dn), (m2_lh, t_lh), (m2_rh, t_rh))):
        z = z_all[:, p * _D:(p + 1) * _D]
        e = jax.lax.dot_general(z, m2cb[...], (((1,), (1,)), ((), ())),
                                precision=_PREC) + cn_all[p, :][None, :]
        m = jnp.min(e, axis=1)
        dmin_total += jnp.sum(m)
        cand = jnp.where(e == m[:, None], iota, jnp.float32(_K))
        idx = jnp.min(cand, axis=1)
        oh = (iota == idx[:, None]).astype(jnp.float32)
        dec.append(jax.lax.dot(oh, tbl[...], precision=_PREC))
    d_up, d_dn, d_lh, d_rh = dec

    xout[...] = jnp.concatenate(
        [d_up[:, :36] + d_dn[:, :36], d_up[:, 36:75], xt[:, 75:165]],
        axis=1)[None]
    xrout[...] = jnp.concatenate(
        [d_up[:, 75:147] + d_dn[:, 36:108], d_up[:, 147:225],
         d_lh, d_rh], axis=1)[None]
    xshift[...] = d_dn[:, 108:111][None]

    @pl.when((pl.program_id(0) == 0) & (pl.program_id(1) == 0))
    def _():
        loss[...] = jnp.zeros_like(loss)
    loss[...] += (1.25 / (n_rows * _D)) * dmin_total.reshape(1, 1)


def kernel(x, xrot, xshift, up_encW, up_encb, up_codebook, up_decW, up_decb,
           down_encW, down_encb, down_codebook, down_decW, down_decb,
           lhand_encW, lhand_encb, lhand_codebook, lhand_decW, lhand_decb,
           rhand_encW, rhand_encb, rhand_codebook, rhand_decW, rhand_decb):
    b, t, c = x.shape
    crot = xrot.shape[-1]
    n = b * t

    wfull = jax.scipy.linalg.block_diag(up_encW, down_encW, lhand_encW,
                                        rhand_encW)
    xrows, rrows, srows = _enc_row_maps()
    wx = wfull[xrows]
    wr = wfull[rrows]
    ws = wfull[srows]
    eb = jnp.concatenate([up_encb, down_encb, lhand_encb,
                          rhand_encb])[None, :]

    up_src, up_msk = _dec_col_map('up', (75, 150))
    dn_src, dn_msk = _dec_col_map('down', (36, 72))
    up_w = up_decW[:, up_src] * up_msk[None, :]
    up_b = (up_decb[up_src] * up_msk)[None, :]
    dn_w = down_decW[:, dn_src] * dn_msk[None, :]
    dn_b = (down_decb[dn_src] * dn_msk)[None, :]
    lh_cp = len(_LH) * _JC
    lh_w = lhand_decW[:, lh_cp:lh_cp + len(_LH) * _RC]
    lh_b = lhand_decb[None, lh_cp:lh_cp + len(_LH) * _RC]
    rh_cp = len(_RH) * _JC
    rh_w = rhand_decW[:, rh_cp:rh_cp + len(_RH) * _RC]
    rh_b = rhand_decb[None, rh_cp:rh_cp + len(_RH) * _RC]

    tbl_shapes = [
        jax.ShapeDtypeStruct((_K, 225), jnp.float32),
        jax.ShapeDtypeStruct((_K, 111), jnp.float32),
        jax.ShapeDtypeStruct((_K, 90), jnp.float32),
        jax.ShapeDtypeStruct((_K, 90), jnp.float32),
        jax.ShapeDtypeStruct((_K, _D), jnp.float32),
        jax.ShapeDtypeStruct((_K, _D), jnp.float32),
        jax.ShapeDtypeStruct((_K, _D), jnp.float32),
        jax.ShapeDtypeStruct((_K, _D), jnp.float32),
        jax.ShapeDtypeStruct((4, _K), jnp.float32),
    ]
    prep = pl.pallas_call(
        _prep_body,
        out_shape=tbl_shapes,
    )(up_codebook, down_codebook, lhand_codebook, rhand_codebook,
      up_w, dn_w, lh_w, rh_w, up_b, dn_b, lh_b, rh_b)
    t_up, t_dn, t_lh, t_rh, m2_up, m2_dn, m2_lh, m2_rh, cn = prep

    tile = 512
    grid = (b, t // tile)
    blk = lambda w: pl.BlockSpec((1, tile, w), lambda i, j: (i, j, 0))
    full = lambda a: pl.BlockSpec(a.shape, lambda i, j: (0,) * a.ndim)

    out_shapes = [
        jax.ShapeDtypeStruct((b, t, c), jnp.float32),
        jax.ShapeDtypeStruct((b, t, crot), jnp.float32),
        jax.ShapeDtypeStruct((b, t, 3), jnp.float32),
        jax.ShapeDtypeStruct((1, 1), jnp.float32),
    ]
    outs = pl.pallas_call(
        functools.partial(_main_body, tile=tile, n_rows=n),
        grid=grid,
        in_specs=[blk(c), blk(crot), blk(3),
                  full(wx), full(wr), full(ws), full(eb),
                  full(m2_up), full(m2_dn), full(m2_lh), full(m2_rh),
                  full(cn),
                  full(t_up), full(t_dn), full(t_lh), full(t_rh)],
        out_specs=[blk(c), blk(crot), blk(3),
                   pl.BlockSpec((1, 1), lambda i, j: (0, 0))],
        out_shape=out_shapes,
    )(x, xrot, xshift, wx, wr, ws, eb,
      m2_up, m2_dn, m2_lh, m2_rh, cn, t_up, t_dn, t_lh, t_rh)
    xout3, xrout3, xshift3, loss = outs

    return (xout3, xrout3, xshift3, loss[0, 0])
